# baseline (speedup 1.0000x reference)
import jax
import jax.numpy as jnp
from jax.experimental import pallas as pl
from jax.experimental.pallas import tpu as pltpu

_N = 20000
_LANES = 128
_ROWS = 160
_CHUNKS = _ROWS // 8
_PAD = _ROWS * _LANES
_MAX_DET = 300
_IOU_THR = 0.45
_CONF_THR = 0.25


def _comb(a, b):
    take_b = (b[0] > a[0]) | ((b[0] == a[0]) & (b[1] < a[1]))
    return tuple(jnp.where(take_b, xb, xa) for xa, xb in zip(a, b))


def _winner(s, flat_ref, x1_ref, y1_ref, x2_ref, y2_ref):
    items = []
    for k in range(_CHUNKS):
        sl = slice(8 * k, 8 * (k + 1))
        items.append((s[sl], flat_ref[sl, :], x1_ref[sl, :], y1_ref[sl, :],
                      x2_ref[sl, :], y2_ref[sl, :]))
    while len(items) > 1:
        nxt = [_comb(a, b) for a, b in zip(items[::2], items[1::2])]
        if len(items) % 2:
            nxt.append(items[-1])
        items = nxt
    w = items[0]
    for sh in (4, 2, 1):
        w = _comb(w, tuple(pltpu.roll(t, sh, axis=0) for t in w))
    for sh in (64, 32, 16, 8, 4, 2, 1):
        w = _comb(w, tuple(pltpu.roll(t, sh, axis=1) for t in w))
    return w


def _nms_kernel(x1_ref, y1_ref, x2_ref, y2_ref, s_ref, out_ref,
                live_ref, area_ref, flat_ref):
    x1 = x1_ref[...]
    y1 = y1_ref[...]
    x2 = x2_ref[...]
    y2 = y2_ref[...]
    area_ref[...] = (x2 - x1) * (y2 - y1)

    row_i = jax.lax.broadcasted_iota(jnp.int32, (_ROWS, _LANES), 0)
    col_i = jax.lax.broadcasted_iota(jnp.int32, (_ROWS, _LANES), 1)
    flat_ref[...] = row_i * _LANES + col_i
    lane_i = jax.lax.broadcasted_iota(jnp.int32, (1, _LANES), 1)

    s0 = s_ref[...]
    s0 = jnp.where(s0 >= _CONF_THR, s0, 0.0)
    live_ref[...] = s0
    w0 = _winner(s0, flat_ref, x1_ref, y1_ref, x2_ref, y2_ref)

    def body(i, w):
        ws = w[0][0:1, :]
        bx1 = w[2][0:1, :]
        by1 = w[3][0:1, :]
        bx2 = w[4][0:1, :]
        by2 = w[5][0:1, :]

        entry = (jnp.where(lane_i == 0, bx1, 0.0)
                 + jnp.where(lane_i == 1, by1, 0.0)
                 + jnp.where(lane_i == 2, bx2, 0.0)
                 + jnp.where(lane_i == 3, by2, 0.0)
                 + jnp.where(lane_i == 4, ws, 0.0))
        out_ref[pl.ds(i, 1), :] = jnp.where(ws > 0.0, entry, 0.0)

        barea = (bx2 - bx1) * (by2 - by1)
        xx1 = jnp.maximum(bx1, x1_ref[...])
        yy1 = jnp.maximum(by1, y1_ref[...])
        xx2 = jnp.minimum(bx2, x2_ref[...])
        yy2 = jnp.minimum(by2, y2_ref[...])
        inter = jnp.maximum(xx2 - xx1, 0.0) * jnp.maximum(yy2 - yy1, 0.0)
        iou = inter / (barea + area_ref[...] - inter + 1e-9)
        s_new = jnp.where(iou > _IOU_THR, 0.0, live_ref[...])
        live_ref[...] = s_new

        return _winner(s_new, flat_ref, x1_ref, y1_ref, x2_ref, y2_ref)

    jax.lax.fori_loop(0, _MAX_DET, body, w0, unroll=False)


def kernel(boxes, scores):
    pb = jnp.pad(boxes, ((0, _PAD - _N), (0, 0)))
    x1 = pb[:, 0].reshape(_ROWS, _LANES)
    y1 = pb[:, 1].reshape(_ROWS, _LANES)
    x2 = pb[:, 2].reshape(_ROWS, _LANES)
    y2 = pb[:, 3].reshape(_ROWS, _LANES)
    s = jnp.pad(scores, (0, _PAD - _N)).reshape(_ROWS, _LANES)

    out = pl.pallas_call(
        _nms_kernel,
        out_shape=jax.ShapeDtypeStruct((_MAX_DET, _LANES), jnp.float32),
        scratch_shapes=[pltpu.VMEM((_ROWS, _LANES), jnp.float32),
                        pltpu.VMEM((_ROWS, _LANES), jnp.float32),
                        pltpu.VMEM((_ROWS, _LANES), jnp.int32)],
    )(x1, y1, x2, y2, s)
    return out[:, :5]


# speedup vs baseline: 2.3813x; 2.3813x over previous
import jax
import jax.numpy as jnp
from jax.experimental import pallas as pl
from jax.experimental.pallas import tpu as pltpu

_N = 20000
_LANES = 128
_ROWS = 160
_CHUNKS = _ROWS // 8
_PAD = _ROWS * _LANES
_BIG = 2 * _PAD
_MAX_DET = 300
_IOU_THR = 0.45
_CONF_THR = 0.25


def _slane_all(v, op):
    for sh in (4, 2, 1):
        v = op(v, pltpu.roll(v, sh, axis=0))
    return v


def _nms_kernel(x1_ref, y1_ref, x2_ref, y2_ref, s_ref, out_ref,
                live_ref, area_ref, flat_ref):
    x1 = x1_ref[...]
    y1 = y1_ref[...]
    x2 = x2_ref[...]
    y2 = y2_ref[...]
    area_ref[...] = (x2 - x1) * (y2 - y1)

    row_i = jax.lax.broadcasted_iota(jnp.int32, (_ROWS, _LANES), 0)
    col_i = jax.lax.broadcasted_iota(jnp.int32, (_ROWS, _LANES), 1)
    flat_ref[...] = col_i * _ROWS + row_i
    lane_i = jax.lax.broadcasted_iota(jnp.int32, (1, _LANES), 1)

    s0 = s_ref[...]
    s0 = jnp.where(s0 >= _CONF_THR, s0, 0.0)
    live_ref[...] = s0

    def _winner(get_s):
        m8 = get_s(0)
        for k in range(1, _CHUNKS):
            m8 = jnp.maximum(m8, get_s(k))
        colmax = _slane_all(m8, jnp.maximum)
        cand = jnp.full((8, _LANES), _BIG, jnp.int32)
        for k in range(_CHUNKS):
            sl = slice(8 * k, 8 * (k + 1))
            hit = get_s(k) == colmax
            cand = jnp.minimum(cand, jnp.where(hit, flat_ref[sl, :], _BIG))
        colidx = _slane_all(cand, jnp.minimum)
        cx1 = jnp.zeros((8, _LANES), jnp.float32)
        cy1 = jnp.zeros((8, _LANES), jnp.float32)
        cx2 = jnp.zeros((8, _LANES), jnp.float32)
        cy2 = jnp.zeros((8, _LANES), jnp.float32)
        for k in range(_CHUNKS):
            sl = slice(8 * k, 8 * (k + 1))
            ex = flat_ref[sl, :] == colidx
            cx1 = cx1 + jnp.where(ex, x1_ref[sl, :], 0.0)
            cy1 = cy1 + jnp.where(ex, y1_ref[sl, :], 0.0)
            cx2 = cx2 + jnp.where(ex, x2_ref[sl, :], 0.0)
            cy2 = cy2 + jnp.where(ex, y2_ref[sl, :], 0.0)
        cx1 = _slane_all(cx1, jnp.add)[0:1, :]
        cy1 = _slane_all(cy1, jnp.add)[0:1, :]
        cx2 = _slane_all(cx2, jnp.add)[0:1, :]
        cy2 = _slane_all(cy2, jnp.add)[0:1, :]
        cm = colmax[0:1, :]
        a = jnp.argmax(cm, axis=1, keepdims=True)
        hitlane = lane_i == a.astype(jnp.int32)
        wm = jnp.sum(jnp.where(hitlane, cm, 0.0), axis=1, keepdims=True)
        wx1 = jnp.sum(jnp.where(hitlane, cx1, 0.0), axis=1, keepdims=True)
        wy1 = jnp.sum(jnp.where(hitlane, cy1, 0.0), axis=1, keepdims=True)
        wx2 = jnp.sum(jnp.where(hitlane, cx2, 0.0), axis=1, keepdims=True)
        wy2 = jnp.sum(jnp.where(hitlane, cy2, 0.0), axis=1, keepdims=True)
        return (wm + jnp.zeros((1, _LANES), jnp.float32),
                wx1 + jnp.zeros((1, _LANES), jnp.float32),
                wy1 + jnp.zeros((1, _LANES), jnp.float32),
                wx2 + jnp.zeros((1, _LANES), jnp.float32),
                wy2 + jnp.zeros((1, _LANES), jnp.float32))

    w0 = _winner(lambda k: live_ref[8 * k:8 * (k + 1), :])

    def body(i, w):
        m, bx1, by1, bx2, by2 = w

        entry = (jnp.where(lane_i == 0, bx1, 0.0)
                 + jnp.where(lane_i == 1, by1, 0.0)
                 + jnp.where(lane_i == 2, bx2, 0.0)
                 + jnp.where(lane_i == 3, by2, 0.0)
                 + jnp.where(lane_i == 4, m, 0.0))
        out_ref[pl.ds(i, 1), :] = jnp.where(m > 0.0, entry, 0.0)

        barea = (bx2 - bx1) * (by2 - by1)
        s_chunks = []
        for k in range(_CHUNKS):
            sl = slice(8 * k, 8 * (k + 1))
            xx1 = jnp.maximum(bx1, x1_ref[sl, :])
            yy1 = jnp.maximum(by1, y1_ref[sl, :])
            xx2 = jnp.minimum(bx2, x2_ref[sl, :])
            yy2 = jnp.minimum(by2, y2_ref[sl, :])
            inter = (jnp.maximum(xx2 - xx1, 0.0)
                     * jnp.maximum(yy2 - yy1, 0.0))
            iou = inter / (barea + area_ref[sl, :] - inter + 1e-9)
            sc = jnp.where(iou > _IOU_THR, 0.0, live_ref[sl, :])
            live_ref[sl, :] = sc
            s_chunks.append(sc)

        return _winner(lambda k: s_chunks[k])

    jax.lax.fori_loop(0, _MAX_DET, body, w0, unroll=False)


def kernel(boxes, scores):
    pb = jnp.pad(boxes, ((0, _PAD - _N), (0, 0)))
    x1 = pb[:, 0].reshape(_LANES, _ROWS).T
    y1 = pb[:, 1].reshape(_LANES, _ROWS).T
    x2 = pb[:, 2].reshape(_LANES, _ROWS).T
    y2 = pb[:, 3].reshape(_LANES, _ROWS).T
    s = jnp.pad(scores, (0, _PAD - _N)).reshape(_LANES, _ROWS).T

    out = pl.pallas_call(
        _nms_kernel,
        out_shape=jax.ShapeDtypeStruct((_MAX_DET, _LANES), jnp.float32),
        scratch_shapes=[pltpu.VMEM((_ROWS, _LANES), jnp.float32),
                        pltpu.VMEM((_ROWS, _LANES), jnp.float32),
                        pltpu.VMEM((_ROWS, _LANES), jnp.int32)],
    )(x1, y1, x2, y2, s)
    return out[:, :5]
